# baseline (speedup 1.0000x reference)
import functools

import jax
import jax.numpy as jnp
from jax import lax
from jax.experimental import pallas as pl
from jax.experimental.pallas import tpu as pltpu
from jax.experimental.pallas import tpu_sc as plsc

N, C, D, GH, GW = 8, 12, 8, 16, 16
H = W = 512
L = 16
NCORES, NSUB = 2, 16
NWORK = NCORES * NSUB
ROWS_PER_W = (N * H) // NWORK
RBLK = 4
NBLK = ROWS_PER_W // RBLK
NCHUNK = W // L
CD = C * D
GXV_PAD = CD * GH + 24


def _splat_i32(s):
    return lax.broadcast_in_dim(jnp.int32(s) if isinstance(s, int) else s,
                                (L,), ())


def _sc_body(grid_hbm, guide_hbm, out_hbm,
             gridv, gxt, gxv, ytab0, fytab, guidebuf, outbuf):
    wid = lax.axis_index("s") * NCORES + lax.axis_index("c")
    n = wid // (NWORK // N)
    rowbase = (wid % (NWORK // N)) * ROWS_PER_W

    pltpu.sync_copy(grid_hbm.at[n], gridv)

    def fill_y(ch, carry):
        wv = lax.iota(jnp.int32, L) + _splat_i32(ch * L)
        t = wv * (GH - 1)
        y0 = lax.div(t, W - 1)
        fy = (t - y0 * (W - 1)).astype(jnp.float32) * (1.0 / (W - 1))
        ytab0[pl.ds(ch * L, L)] = y0
        fytab[pl.ds(ch * L, L)] = fy
        return carry
    lax.fori_loop(0, NCHUNK, fill_y, 0)

    yiota = lax.iota(jnp.int32, L)

    def build_t(cd, carry):
        cv = _splat_i32(lax.div(cd, D))
        dv = _splat_i32(lax.rem(cd, D))
        for x in range(GW):
            col = plsc.load_gather(gridv, [cv, dv, yiota, _splat_i32(x)])
            gxt[pl.ds(cd * (GW * GH) + x * GH, GH)] = col
        return carry
    lax.fori_loop(0, CD, build_t, 0)

    zpad = jnp.zeros((L,), jnp.float32)
    gxv[pl.ds(CD * GH, L)] = zpad
    gxv[pl.ds(GXV_PAD - L, L)] = zpad

    def do_block(blk, carry):
        h0 = rowbase + blk * RBLK
        pltpu.sync_copy(guide_hbm.at[n, 0, pl.ds(h0, RBLK)], guidebuf)

        def do_row(rr, carry):
            h = h0 + rr
            t = h * (GW - 1)
            x0 = t // (H - 1)
            fx = (t - x0 * (H - 1)).astype(jnp.float32) * (1.0 / (H - 1))
            x1 = jnp.minimum(x0 + 1, GW - 1)
            fxv = lax.broadcast_in_dim(fx, (L,), ())
            fxc = 1.0 - fxv
            xoff = x0 * GH
            dx = (x1 - x0) * GH

            @plsc.parallel_loop(0, CD, unroll=4)
            def fold_x(cd):
                b0 = cd * (GW * GH) + xoff
                v0 = gxt[pl.ds(b0, GH)]
                v1 = gxt[pl.ds(b0 + dx, GH)]
                gxv[pl.ds(cd * GH, GH)] = v0 * fxc + v1 * fxv

            @plsc.parallel_loop(0, NCHUNK, unroll=4)
            def do_chunk(ch):
                g = guidebuf[rr, pl.ds(ch * L, L)]
                z = jnp.minimum(jnp.maximum(g * 3.5 + 3.5, 0.0), float(D - 1))
                z0 = jnp.minimum(z.astype(jnp.int32), D - 2)
                fz = z - z0.astype(jnp.float32)
                y0 = ytab0[pl.ds(ch * L, L)]
                fy = fytab[pl.ds(ch * L, L)]
                wz0 = 1.0 - fz
                wy0 = 1.0 - fy
                w00 = wz0 * wy0
                w01 = wz0 * fy
                w10 = fz * wy0
                w11 = fz * fy
                ib0 = z0 * GH + y0
                ib1 = ib0 + 1
                for c in range(C):
                    o = c * (D * GH)
                    a00 = plsc.load_gather(gxv.at[pl.ds(o, 128)], [ib0])
                    a01 = plsc.load_gather(gxv.at[pl.ds(o, 128)], [ib1])
                    a10 = plsc.load_gather(gxv.at[pl.ds(o + GH, 128)], [ib0])
                    a11 = plsc.load_gather(gxv.at[pl.ds(o + GH, 128)], [ib1])
                    res = a00 * w00 + a01 * w01 + a10 * w10 + a11 * w11
                    outbuf[c, rr, pl.ds(ch * L, L)] = res
            return carry
        lax.fori_loop(0, RBLK, do_row, 0)

        pltpu.sync_copy(outbuf, out_hbm.at[n, :, pl.ds(h0, RBLK)])
        return carry
    lax.fori_loop(0, NBLK, do_block, 0)


_SCRATCH = [
    pltpu.VMEM((C, D, GH, GW), jnp.float32),
    pltpu.VMEM((CD * GW * GH,), jnp.float32),
    pltpu.VMEM((GXV_PAD,), jnp.float32),
    pltpu.VMEM((W,), jnp.int32),
    pltpu.VMEM((W,), jnp.float32),
    pltpu.VMEM((RBLK, W), jnp.float32),
    pltpu.VMEM((C, RBLK, W), jnp.float32),
]

kernel = functools.partial(
    pl.kernel,
    out_type=jax.ShapeDtypeStruct((N, C, H, W), jnp.float32),
    mesh=plsc.VectorSubcoreMesh(core_axis_name="c", subcore_axis_name="s"),
    scratch_types=_SCRATCH,
    compiler_params=pltpu.CompilerParams(needs_layout_passes=False,
                                         use_tc_tiling_on_sc=False),
)(_sc_body)


# speedup vs baseline: 1.0927x; 1.0927x over previous
import functools

import jax
import jax.numpy as jnp
from jax import lax
from jax.experimental import pallas as pl
from jax.experimental.pallas import tpu as pltpu
from jax.experimental.pallas import tpu_sc as plsc

N, C, D, GH, GW = 8, 12, 8, 16, 16
H = W = 512
L = 16
NCORES, NSUB = 2, 16
NWORK = NCORES * NSUB
ROWS_PER_W = (N * H) // NWORK
RBLK = 4
NBLK = ROWS_PER_W // RBLK
NCHUNK = W // L
CD = C * D
GXV_PAD = CD * GH + 24


def _splat_i32(s):
    return lax.broadcast_in_dim(jnp.int32(s) if isinstance(s, int) else s,
                                (L,), ())


def _sc_body(grid_hbm, guide_hbm, out_hbm,
             gridv, gxt, gxv, ytab0, fytab, guidebuf, outbuf,
             gsem0, gsem1, osem0, osem1):
    wid = lax.axis_index("s") * NCORES + lax.axis_index("c")
    n = wid // (NWORK // N)
    rowbase = (wid % (NWORK // N)) * ROWS_PER_W

    pltpu.sync_copy(grid_hbm.at[n], gridv)

    def fill_y(ch, carry):
        wv = lax.iota(jnp.int32, L) + _splat_i32(ch * L)
        t = wv * (GH - 1)
        y0 = lax.div(t, W - 1)
        fy = (t - y0 * (W - 1)).astype(jnp.float32) * (1.0 / (W - 1))
        ytab0[pl.ds(ch * L, L)] = y0
        fytab[pl.ds(ch * L, L)] = fy
        return carry
    lax.fori_loop(0, NCHUNK, fill_y, 0)

    yiota = lax.iota(jnp.int32, L)

    def build_t(cd, carry):
        cv = _splat_i32(lax.div(cd, D))
        dv = _splat_i32(lax.rem(cd, D))
        for x in range(GW):
            col = plsc.load_gather(gridv, [cv, dv, yiota, _splat_i32(x)])
            gxt[pl.ds(cd * (GW * GH) + x * GH, GH)] = col
        return carry
    lax.fori_loop(0, CD, build_t, 0)

    zpad = jnp.zeros((L,), jnp.float32)
    gxv[pl.ds(CD * GH, L)] = zpad
    gxv[pl.ds(GXV_PAD - L, L)] = zpad

    def _guide_src(blk):
        return guide_hbm.at[n, 0, pl.ds(rowbase + blk * RBLK, RBLK)]

    def _out_dst(blk):
        return out_hbm.at[n, :, pl.ds(rowbase + blk * RBLK, RBLK)]

    def _compute_block(blk, gbuf, obuf):
        h0 = rowbase + blk * RBLK

        def do_row(rr, carry):
            h = h0 + rr
            t = h * (GW - 1)
            x0 = t // (H - 1)
            fx = (t - x0 * (H - 1)).astype(jnp.float32) * (1.0 / (H - 1))
            x1 = jnp.minimum(x0 + 1, GW - 1)
            fxv = lax.broadcast_in_dim(fx, (L,), ())
            fxc = 1.0 - fxv
            xoff = x0 * GH
            dx = (x1 - x0) * GH

            @plsc.parallel_loop(0, CD, unroll=2)
            def fold_x(cd):
                b0 = cd * (GW * GH) + xoff
                v0 = gxt[pl.ds(b0, GH)]
                v1 = gxt[pl.ds(b0 + dx, GH)]
                gxv[pl.ds(cd * GH, GH)] = v0 * fxc + v1 * fxv

            @plsc.parallel_loop(0, NCHUNK, unroll=2)
            def do_chunk(ch):
                g = gbuf[rr, pl.ds(ch * L, L)]
                z = jnp.minimum(jnp.maximum(g * 3.5 + 3.5, 0.0), float(D - 1))
                z0 = jnp.minimum(z.astype(jnp.int32), D - 2)
                fz = z - z0.astype(jnp.float32)
                y0 = ytab0[pl.ds(ch * L, L)]
                fy = fytab[pl.ds(ch * L, L)]
                wz0 = 1.0 - fz
                wy0 = 1.0 - fy
                w00 = wz0 * wy0
                w01 = wz0 * fy
                w10 = fz * wy0
                w11 = fz * fy
                ib0 = z0 * GH + y0
                ib1 = ib0 + 1
                for c in range(C):
                    o = c * (D * GH)
                    a00 = plsc.load_gather(gxv.at[pl.ds(o, 128)], [ib0])
                    a01 = plsc.load_gather(gxv.at[pl.ds(o, 128)], [ib1])
                    a10 = plsc.load_gather(gxv.at[pl.ds(o + GH, 128)], [ib0])
                    a11 = plsc.load_gather(gxv.at[pl.ds(o + GH, 128)], [ib1])
                    res = a00 * w00 + a01 * w01 + a10 * w10 + a11 * w11
                    obuf[c, rr, pl.ds(ch * L, L)] = res
            return carry
        lax.fori_loop(0, RBLK, do_row, 0)

    gb0, gb1 = guidebuf.at[0], guidebuf.at[1]
    ob0, ob1 = outbuf.at[0], outbuf.at[1]
    pltpu.async_copy(_guide_src(0), gb0, gsem0)

    def do_pair(bp, carry):
        blk0 = 2 * bp
        blk1 = blk0 + 1
        pltpu.async_copy(_guide_src(blk1), gb1, gsem1)
        pltpu.make_async_copy(_guide_src(blk0), gb0, gsem0).wait()

        @pl.when(bp > 0)
        def _():
            pltpu.make_async_copy(ob0, _out_dst(blk0), osem0).wait()
        _compute_block(blk0, gb0, ob0)
        pltpu.async_copy(ob0, _out_dst(blk0), osem0)

        @pl.when(bp + 1 < NBLK // 2)
        def _():
            pltpu.async_copy(_guide_src(blk0 + 2), gb0, gsem0)
        pltpu.make_async_copy(_guide_src(blk1), gb1, gsem1).wait()

        @pl.when(bp > 0)
        def _():
            pltpu.make_async_copy(ob1, _out_dst(blk1), osem1).wait()
        _compute_block(blk1, gb1, ob1)
        pltpu.async_copy(ob1, _out_dst(blk1), osem1)
        return carry
    lax.fori_loop(0, NBLK // 2, do_pair, 0)

    pltpu.make_async_copy(ob0, _out_dst(NBLK - 2), osem0).wait()
    pltpu.make_async_copy(ob1, _out_dst(NBLK - 1), osem1).wait()


_SCRATCH = [
    pltpu.VMEM((C, D, GH, GW), jnp.float32),
    pltpu.VMEM((CD * GW * GH,), jnp.float32),
    pltpu.VMEM((GXV_PAD,), jnp.float32),
    pltpu.VMEM((W,), jnp.int32),
    pltpu.VMEM((W,), jnp.float32),
    pltpu.VMEM((2, RBLK, W), jnp.float32),
    pltpu.VMEM((2, C, RBLK, W), jnp.float32),
    pltpu.SemaphoreType.DMA,
    pltpu.SemaphoreType.DMA,
    pltpu.SemaphoreType.DMA,
    pltpu.SemaphoreType.DMA,
]

kernel = functools.partial(
    pl.kernel,
    out_type=jax.ShapeDtypeStruct((N, C, H, W), jnp.float32),
    mesh=plsc.VectorSubcoreMesh(core_axis_name="c", subcore_axis_name="s"),
    scratch_types=_SCRATCH,
    compiler_params=pltpu.CompilerParams(needs_layout_passes=False,
                                         use_tc_tiling_on_sc=False),
)(_sc_body)


# speedup vs baseline: 2.5958x; 2.3755x over previous
import functools

import jax
import jax.numpy as jnp
from jax import lax
from jax.experimental import pallas as pl
from jax.experimental.pallas import tpu as pltpu
from jax.experimental.pallas import tpu_sc as plsc

N, C, D, GH, GW = 8, 12, 8, 16, 16
H = W = 512
L = 16
NCORES, NSUB = 2, 16
NWORK = NCORES * NSUB
ROWS_PER_W = (N * H) // NWORK
RBLK = 4
NBLK = ROWS_PER_W // RBLK
NCHUNK = W // L
CD = C * D
GXV_PAD = CD * GH + 24


def _splat_i32(s):
    return lax.broadcast_in_dim(jnp.int32(s) if isinstance(s, int) else s,
                                (L,), ())


def _sc_body(grid_hbm, guide_hbm, out_hbm,
             gridv, gxt, gxv, ytab0, fytab, guidebuf, outbuf,
             gsem0, gsem1, osem0, osem1):
    wid = lax.axis_index("s") * NCORES + lax.axis_index("c")
    n = wid // (NWORK // N)
    rowbase = (wid % (NWORK // N)) * ROWS_PER_W

    pltpu.sync_copy(grid_hbm.at[n], gridv)

    def fill_y(ch, carry):
        wv = lax.iota(jnp.int32, L) + _splat_i32(ch * L)
        t = wv * (GH - 1)
        y0 = lax.div(t, W - 1)
        fy = (t - y0 * (W - 1)).astype(jnp.float32) * (1.0 / (W - 1))
        ytab0[pl.ds(ch * L, L)] = y0 * D
        fytab[pl.ds(ch * L, L)] = fy
        return carry
    lax.fori_loop(0, NCHUNK, fill_y, 0)

    yiota = lax.iota(jnp.int32, L)
    d_lane = lax.rem(yiota, _splat_i32(D))
    yh_lane = lax.div(yiota, _splat_i32(D))

    def build_t(cx, carry):
        cv = _splat_i32(lax.div(cx, GW))
        xv = _splat_i32(lax.rem(cx, GW))
        for yp in range(GH // 2):
            ylane = yh_lane + _splat_i32(2 * yp)
            col = plsc.load_gather(gridv, [cv, d_lane, ylane, xv])
            gxt[pl.ds(cx * (GH * D) + yp * L, L)] = col
        return carry
    lax.fori_loop(0, C * GW, build_t, 0)

    zpad = jnp.zeros((L,), jnp.float32)
    gxv[pl.ds(CD * GH, L)] = zpad
    gxv[pl.ds(GXV_PAD - L, L)] = zpad

    def _guide_src(blk):
        return guide_hbm.at[n, 0, pl.ds(rowbase + blk * RBLK, RBLK)]

    def _out_dst(blk):
        return out_hbm.at[n, :, pl.ds(rowbase + blk * RBLK, RBLK)]

    def _compute_block(blk, gbuf, obuf):
        h0 = rowbase + blk * RBLK

        def do_row(rr, carry):
            h = h0 + rr
            t = h * (GW - 1)
            x0 = t // (H - 1)
            fx = (t - x0 * (H - 1)).astype(jnp.float32) * (1.0 / (H - 1))
            x1 = jnp.minimum(x0 + 1, GW - 1)
            fxv = lax.broadcast_in_dim(fx, (L,), ())
            fxc = 1.0 - fxv
            xoff = x0 * (GH * D)
            dx = (x1 - x0) * (GH * D)

            @plsc.parallel_loop(0, C * (GH // 2), unroll=2)
            def fold_x(e):
                c = lax.div(e, GH // 2)
                yp = lax.rem(e, GH // 2)
                b0 = c * (GW * GH * D) + xoff + yp * L
                v0 = gxt[pl.ds(b0, L)]
                v1 = gxt[pl.ds(b0 + dx, L)]
                gxv[pl.ds(e * L, L)] = v0 * fxc + v1 * fxv

            @plsc.parallel_loop(0, NCHUNK, unroll=2)
            def do_chunk(ch):
                g = gbuf[rr, pl.ds(ch * L, L)]
                z = jnp.minimum(jnp.maximum(g * 3.5 + 3.5, 0.0), float(D - 1))
                z0 = jnp.minimum(z.astype(jnp.int32), D - 2)
                fz = z - z0.astype(jnp.float32)
                y0 = ytab0[pl.ds(ch * L, L)]
                fy = fytab[pl.ds(ch * L, L)]
                wz0 = 1.0 - fz
                wy0 = 1.0 - fy
                w00 = wz0 * wy0
                w01 = wz0 * fy
                w10 = fz * wy0
                w11 = fz * fy
                ib0 = z0 + y0
                ib1 = ib0 + 1
                for c in range(C):
                    o = c * (D * GH)
                    a00 = plsc.load_gather(gxv.at[pl.ds(o, 128)], [ib0])
                    a10 = plsc.load_gather(gxv.at[pl.ds(o, 128)], [ib1])
                    a01 = plsc.load_gather(gxv.at[pl.ds(o + D, 128)], [ib0])
                    a11 = plsc.load_gather(gxv.at[pl.ds(o + D, 128)], [ib1])
                    res = a00 * w00 + a01 * w01 + a10 * w10 + a11 * w11
                    obuf[c, rr, pl.ds(ch * L, L)] = res
            return carry
        lax.fori_loop(0, RBLK, do_row, 0)

    gb0, gb1 = guidebuf.at[0], guidebuf.at[1]
    ob0, ob1 = outbuf.at[0], outbuf.at[1]
    pltpu.async_copy(_guide_src(0), gb0, gsem0)

    def do_pair(bp, carry):
        blk0 = 2 * bp
        blk1 = blk0 + 1
        pltpu.async_copy(_guide_src(blk1), gb1, gsem1)
        pltpu.make_async_copy(_guide_src(blk0), gb0, gsem0).wait()

        @pl.when(bp > 0)
        def _():
            pltpu.make_async_copy(ob0, _out_dst(blk0), osem0).wait()
        _compute_block(blk0, gb0, ob0)
        pltpu.async_copy(ob0, _out_dst(blk0), osem0)

        @pl.when(bp + 1 < NBLK // 2)
        def _():
            pltpu.async_copy(_guide_src(blk0 + 2), gb0, gsem0)
        pltpu.make_async_copy(_guide_src(blk1), gb1, gsem1).wait()

        @pl.when(bp > 0)
        def _():
            pltpu.make_async_copy(ob1, _out_dst(blk1), osem1).wait()
        _compute_block(blk1, gb1, ob1)
        pltpu.async_copy(ob1, _out_dst(blk1), osem1)
        return carry
    lax.fori_loop(0, NBLK // 2, do_pair, 0)

    pltpu.make_async_copy(ob0, _out_dst(NBLK - 2), osem0).wait()
    pltpu.make_async_copy(ob1, _out_dst(NBLK - 1), osem1).wait()


_SCRATCH = [
    pltpu.VMEM((C, D, GH, GW), jnp.float32),
    pltpu.VMEM((CD * GW * GH,), jnp.float32),
    pltpu.VMEM((GXV_PAD,), jnp.float32),
    pltpu.VMEM((W,), jnp.int32),
    pltpu.VMEM((W,), jnp.float32),
    pltpu.VMEM((2, RBLK, W), jnp.float32),
    pltpu.VMEM((2, C, RBLK, W), jnp.float32),
    pltpu.SemaphoreType.DMA,
    pltpu.SemaphoreType.DMA,
    pltpu.SemaphoreType.DMA,
    pltpu.SemaphoreType.DMA,
]

kernel = functools.partial(
    pl.kernel,
    out_type=jax.ShapeDtypeStruct((N, C, H, W), jnp.float32),
    mesh=plsc.VectorSubcoreMesh(core_axis_name="c", subcore_axis_name="s"),
    scratch_types=_SCRATCH,
    compiler_params=pltpu.CompilerParams(needs_layout_passes=False,
                                         use_tc_tiling_on_sc=False),
)(_sc_body)


# speedup vs baseline: 2.6078x; 1.0046x over previous
import functools

import jax
import jax.numpy as jnp
from jax import lax
from jax.experimental import pallas as pl
from jax.experimental.pallas import tpu as pltpu
from jax.experimental.pallas import tpu_sc as plsc

N, C, D, GH, GW = 8, 12, 8, 16, 16
H = W = 512
L = 16
NCORES, NSUB = 2, 16
NWORK = NCORES * NSUB
ROWS_PER_W = (N * H) // NWORK
RBLK = 4
NBLK = ROWS_PER_W // RBLK
NCHUNK = W // L
CD = C * D
GXB = CD * GH
COPY_OFF = GXB + 8
GXV_PAD = COPY_OFF + GXB + 24


def _splat_i32(s):
    return lax.broadcast_in_dim(jnp.int32(s) if isinstance(s, int) else s,
                                (L,), ())


def _sc_body(grid_hbm, guide_hbm, out_hbm,
             gridv, gxt, gxv, ytab0, fytab, guidebuf, outbuf,
             gsem0, gsem1, osem0, osem1):
    wid = lax.axis_index("s") * NCORES + lax.axis_index("c")
    n = wid // (NWORK // N)
    rowbase = (wid % (NWORK // N)) * ROWS_PER_W

    pltpu.sync_copy(grid_hbm.at[n], gridv)

    def fill_y(ch, carry):
        wv = lax.iota(jnp.int32, L) + _splat_i32(ch * L)
        t = wv * (GH - 1)
        y0 = lax.div(t, W - 1)
        fy = (t - y0 * (W - 1)).astype(jnp.float32) * (1.0 / (W - 1))
        ytab0[pl.ds(ch * L, L)] = y0 * D + lane_copy_off
        fytab[pl.ds(ch * L, L)] = fy
        return carry

    yiota = lax.iota(jnp.int32, L)
    d_lane = lax.rem(yiota, _splat_i32(D))
    yh_lane = lax.div(yiota, _splat_i32(D))
    lane_copy_off = yh_lane * COPY_OFF
    lax.fori_loop(0, NCHUNK, fill_y, 0)

    def build_t(cx, carry):
        cv = _splat_i32(lax.div(cx, GW))
        xv = _splat_i32(lax.rem(cx, GW))
        for yp in range(GH // 2):
            ylane = yh_lane + _splat_i32(2 * yp)
            col = plsc.load_gather(gridv, [cv, d_lane, ylane, xv])
            gxt[pl.ds(cx * (GH * D) + yp * L, L)] = col
        return carry
    lax.fori_loop(0, C * GW, build_t, 0)

    zpad = jnp.zeros((L,), jnp.float32)
    gxv[pl.ds(GXB, L)] = zpad
    gxv[pl.ds(COPY_OFF + GXB, L)] = zpad
    gxv[pl.ds(GXV_PAD - L, L)] = zpad

    def _guide_src(blk):
        return guide_hbm.at[n, 0, pl.ds(rowbase + blk * RBLK, RBLK)]

    def _out_dst(blk):
        return out_hbm.at[n, :, pl.ds(rowbase + blk * RBLK, RBLK)]

    def _compute_block(blk, gbuf, obuf):
        h0 = rowbase + blk * RBLK

        def do_row(rr, carry):
            h = h0 + rr
            t = h * (GW - 1)
            x0 = t // (H - 1)
            fx = (t - x0 * (H - 1)).astype(jnp.float32) * (1.0 / (H - 1))
            x1 = jnp.minimum(x0 + 1, GW - 1)
            fxv = lax.broadcast_in_dim(fx, (L,), ())
            fxc = 1.0 - fxv
            xoff = x0 * (GH * D)
            dx = (x1 - x0) * (GH * D)

            @plsc.parallel_loop(0, C * (GH // 2), unroll=2)
            def fold_x(e):
                c = lax.div(e, GH // 2)
                yp = lax.rem(e, GH // 2)
                b0 = c * (GW * GH * D) + xoff + yp * L
                v0 = gxt[pl.ds(b0, L)]
                v1 = gxt[pl.ds(b0 + dx, L)]
                gx = v0 * fxc + v1 * fxv
                gxv[pl.ds(e * L, L)] = gx
                gxv[pl.ds(COPY_OFF + e * L, L)] = gx

            @plsc.parallel_loop(0, NCHUNK, unroll=2)
            def do_chunk(ch):
                g = gbuf[rr, pl.ds(ch * L, L)]
                z = jnp.minimum(jnp.maximum(g * 3.5 + 3.5, 0.0), float(D - 1))
                z0 = jnp.minimum(z.astype(jnp.int32), D - 2)
                fz = z - z0.astype(jnp.float32)
                y0 = ytab0[pl.ds(ch * L, L)]
                fy = fytab[pl.ds(ch * L, L)]
                wz0 = 1.0 - fz
                wy0 = 1.0 - fy
                w00 = wz0 * wy0
                w01 = wz0 * fy
                w10 = fz * wy0
                w11 = fz * fy
                ib0 = z0 + y0
                ib1 = ib0 + 1
                span = COPY_OFF + 128
                for c in range(C):
                    o = c * (D * GH)
                    a00 = plsc.load_gather(gxv.at[pl.ds(o, span)], [ib0])
                    a10 = plsc.load_gather(gxv.at[pl.ds(o, span)], [ib1])
                    a01 = plsc.load_gather(gxv.at[pl.ds(o + D, span)], [ib0])
                    a11 = plsc.load_gather(gxv.at[pl.ds(o + D, span)], [ib1])
                    res = a00 * w00 + a01 * w01 + a10 * w10 + a11 * w11
                    obuf[c, rr, pl.ds(ch * L, L)] = res
            return carry
        lax.fori_loop(0, RBLK, do_row, 0)

    gb0, gb1 = guidebuf.at[0], guidebuf.at[1]
    ob0, ob1 = outbuf.at[0], outbuf.at[1]
    pltpu.async_copy(_guide_src(0), gb0, gsem0)

    def do_pair(bp, carry):
        blk0 = 2 * bp
        blk1 = blk0 + 1
        pltpu.async_copy(_guide_src(blk1), gb1, gsem1)
        pltpu.make_async_copy(_guide_src(blk0), gb0, gsem0).wait()

        @pl.when(bp > 0)
        def _():
            pltpu.make_async_copy(ob0, _out_dst(blk0), osem0).wait()
        _compute_block(blk0, gb0, ob0)
        pltpu.async_copy(ob0, _out_dst(blk0), osem0)

        @pl.when(bp + 1 < NBLK // 2)
        def _():
            pltpu.async_copy(_guide_src(blk0 + 2), gb0, gsem0)
        pltpu.make_async_copy(_guide_src(blk1), gb1, gsem1).wait()

        @pl.when(bp > 0)
        def _():
            pltpu.make_async_copy(ob1, _out_dst(blk1), osem1).wait()
        _compute_block(blk1, gb1, ob1)
        pltpu.async_copy(ob1, _out_dst(blk1), osem1)
        return carry
    lax.fori_loop(0, NBLK // 2, do_pair, 0)

    pltpu.make_async_copy(ob0, _out_dst(NBLK - 2), osem0).wait()
    pltpu.make_async_copy(ob1, _out_dst(NBLK - 1), osem1).wait()


_SCRATCH = [
    pltpu.VMEM((C, D, GH, GW), jnp.float32),
    pltpu.VMEM((CD * GW * GH,), jnp.float32),
    pltpu.VMEM((GXV_PAD,), jnp.float32),
    pltpu.VMEM((W,), jnp.int32),
    pltpu.VMEM((W,), jnp.float32),
    pltpu.VMEM((2, RBLK, W), jnp.float32),
    pltpu.VMEM((2, C, RBLK, W), jnp.float32),
    pltpu.SemaphoreType.DMA,
    pltpu.SemaphoreType.DMA,
    pltpu.SemaphoreType.DMA,
    pltpu.SemaphoreType.DMA,
]

kernel = functools.partial(
    pl.kernel,
    out_type=jax.ShapeDtypeStruct((N, C, H, W), jnp.float32),
    mesh=plsc.VectorSubcoreMesh(core_axis_name="c", subcore_axis_name="s"),
    scratch_types=_SCRATCH,
    compiler_params=pltpu.CompilerParams(needs_layout_passes=False,
                                         use_tc_tiling_on_sc=False),
)(_sc_body)
